# baseline (speedup 1.0000x reference)
import functools

import jax
import jax.numpy as jnp
from jax import lax
from jax.experimental import pallas as pl
from jax.experimental.pallas import tpu as pltpu
from jax.experimental.pallas import tpu_sc as plsc

VOCAB_SIZE = 100000
D_DIM = 16
B_DIM = 1024

_INFO = plsc.get_sparse_core_info()
_NC = _INFO.num_cores
_NW = _INFO.num_cores * _INFO.num_subcores
_BPW = B_DIM // _NW

_MESH = plsc.VectorSubcoreMesh(core_axis_name="c", subcore_axis_name="s")


@functools.partial(
    pl.kernel,
    mesh=_MESH,
    out_type=jax.ShapeDtypeStruct((B_DIM, D_DIM), jnp.bfloat16),
    scratch_types=[
        pltpu.VMEM((_BPW,), jnp.int32),
        pltpu.VMEM((_BPW, D_DIM), jnp.bfloat16),
        pltpu.SemaphoreType.DMA,
    ],
    compiler_params=pltpu.CompilerParams(use_tc_tiling_on_sc=False),
)
def _sc_gather(idx_hbm, table_hbm, out_hbm, idx_v, rows_v, sem):
    wid = lax.axis_index("s") * _NC + lax.axis_index("c")
    base = wid * _BPW
    pltpu.sync_copy(idx_hbm.at[pl.ds(base, _BPW)], idx_v)
    pltpu.async_copy(table_hbm.at[idx_v], rows_v, sem).wait()
    pltpu.sync_copy(rows_v, out_hbm.at[pl.ds(base, _BPW)])


_VB = 2048
_NBLK = (VOCAB_SIZE + _VB - 1) // _VB
_TAIL = VOCAB_SIZE - (_NBLK - 1) * _VB
_K = 4


def _mm_body(wt_ref, emb_ref, out_ref, ring, sems):
    i = pl.program_id(0)
    s = lax.rem(i, _K)

    @pl.when(i >= _K)
    def _wait_prev():
        pltpu.make_async_copy(
            ring.at[s],
            out_ref.at[pl.ds((i - _K) * _VB, _VB), :],
            sems.at[s],
        ).wait()

    ring[s] = lax.dot_general(
        wt_ref[...],
        emb_ref[...],
        dimension_numbers=(((0,), (1,)), ((), ())),
        preferred_element_type=jnp.float32,
    )

    @pl.when(i < _NBLK - 1)
    def _start_full():
        pltpu.make_async_copy(
            ring.at[s],
            out_ref.at[pl.ds(i * _VB, _VB), :],
            sems.at[s],
        ).start()

    @pl.when(i == _NBLK - 1)
    def _start_tail_and_drain():
        pltpu.make_async_copy(
            ring.at[s, : _TAIL, :],
            out_ref.at[pl.ds((_NBLK - 1) * _VB, _TAIL), :],
            sems.at[s],
        ).start()
        for j in range(_K):
            t = _NBLK - _K + j
            if t == _NBLK - 1:
                pltpu.make_async_copy(
                    ring.at[t % _K, : _TAIL, :],
                    out_ref.at[pl.ds(t * _VB, _TAIL), :],
                    sems.at[t % _K],
                ).wait()
            else:
                pltpu.make_async_copy(
                    ring.at[t % _K],
                    out_ref.at[pl.ds(t * _VB, _VB), :],
                    sems.at[t % _K],
                ).wait()


def _project_t(Wt, emb):
    return pl.pallas_call(
        _mm_body,
        grid=(_NBLK,),
        in_specs=[
            pl.BlockSpec((D_DIM, _VB), lambda i: (0, i)),
            pl.BlockSpec((B_DIM, D_DIM), lambda i: (0, 0)),
        ],
        out_specs=pl.BlockSpec(memory_space=pltpu.MemorySpace.HBM),
        out_shape=jax.ShapeDtypeStruct((VOCAB_SIZE, B_DIM), jnp.float32),
        scratch_shapes=[
            pltpu.VMEM((_K, _VB, B_DIM), jnp.float32),
            pltpu.SemaphoreType.DMA((_K,)),
        ],
        compiler_params=pltpu.CompilerParams(
            vmem_limit_bytes=100 * 1024 * 1024,
        ),
    )(Wt, emb)


def kernel(indices, table, W):
    emb16 = _sc_gather(indices.astype(jnp.int32), table.astype(jnp.bfloat16))
    return _project_t(W.T, emb16.astype(jnp.float32)).T


# speedup vs baseline: 1.3069x; 1.3069x over previous
import functools

import jax
import jax.numpy as jnp
from jax import lax
from jax.experimental import pallas as pl
from jax.experimental.pallas import tpu as pltpu
from jax.experimental.pallas import tpu_sc as plsc

VOCAB_SIZE = 100000
D_DIM = 16
B_DIM = 1024

_INFO = plsc.get_sparse_core_info()
_NC = _INFO.num_cores
_NW = _INFO.num_cores * _INFO.num_subcores
_BPW = B_DIM // _NW

_MESH = plsc.VectorSubcoreMesh(core_axis_name="c", subcore_axis_name="s")


_GCHUNK = 128
_EPW = _BPW * D_DIM
_L = _INFO.num_lanes


@functools.partial(
    pl.kernel,
    mesh=_MESH,
    out_type=jax.ShapeDtypeStruct((D_DIM, B_DIM), jnp.float32),
    scratch_types=[
        pltpu.VMEM((_BPW,), jnp.int32),
        pltpu.VMEM((_EPW,), jnp.int32),
        pltpu.VMEM((_EPW,), jnp.float32),
        pltpu.SemaphoreType.DMA,
    ],
    compiler_params=pltpu.CompilerParams(use_tc_tiling_on_sc=False),
)
def _sc_gather(idx_hbm, tflat_hbm, out_hbm, idx_v, eidx_v, rows_v, sem):
    wid = lax.axis_index("s") * _NC + lax.axis_index("c")
    base = wid * _BPW
    pltpu.sync_copy(idx_hbm.at[pl.ds(base, _BPW)], idx_v)
    for d in range(D_DIM):
        for h in range(_BPW // _L):
            eidx_v[pl.ds(d * _BPW + h * _L, _L)] = (
                idx_v[pl.ds(h * _L, _L)] + d * VOCAB_SIZE
            )
    copies = []
    for k in range(_EPW // _GCHUNK):
        copies.append(
            pltpu.async_copy(
                tflat_hbm.at[eidx_v.at[pl.ds(k * _GCHUNK, _GCHUNK)]],
                rows_v.at[pl.ds(k * _GCHUNK, _GCHUNK)],
                sem,
            )
        )
    for c in copies:
        c.wait()
    for d in range(D_DIM):
        pltpu.sync_copy(
            rows_v.at[pl.ds(d * _BPW, _BPW)],
            out_hbm.at[d, pl.ds(base, _BPW)],
        )


_VB = 2048
_NBLK = (VOCAB_SIZE + _VB - 1) // _VB
_TAIL = VOCAB_SIZE - (_NBLK - 1) * _VB
_K = 4


def _mm_body(wt_ref, embt_ref, out_ref, ring, sems):
    i = pl.program_id(0)
    s = lax.rem(i, _K)

    @pl.when(i >= _K)
    def _wait_prev():
        pltpu.make_async_copy(
            ring.at[s],
            out_ref.at[pl.ds((i - _K) * _VB, _VB), :],
            sems.at[s],
        ).wait()

    ring[s] = lax.dot_general(
        wt_ref[...],
        embt_ref[...],
        dimension_numbers=(((0,), (0,)), ((), ())),
        preferred_element_type=jnp.float32,
    )

    @pl.when(i < _NBLK - 1)
    def _start_full():
        pltpu.make_async_copy(
            ring.at[s],
            out_ref.at[pl.ds(i * _VB, _VB), :],
            sems.at[s],
        ).start()

    @pl.when(i == _NBLK - 1)
    def _start_tail_and_drain():
        pltpu.make_async_copy(
            ring.at[s, : _TAIL, :],
            out_ref.at[pl.ds((_NBLK - 1) * _VB, _TAIL), :],
            sems.at[s],
        ).start()
        for j in range(_K):
            t = _NBLK - _K + j
            if t == _NBLK - 1:
                pltpu.make_async_copy(
                    ring.at[t % _K, : _TAIL, :],
                    out_ref.at[pl.ds(t * _VB, _TAIL), :],
                    sems.at[t % _K],
                ).wait()
            else:
                pltpu.make_async_copy(
                    ring.at[t % _K],
                    out_ref.at[pl.ds(t * _VB, _VB), :],
                    sems.at[t % _K],
                ).wait()


def _project_t(Wt, embt):
    return pl.pallas_call(
        _mm_body,
        grid=(_NBLK,),
        in_specs=[
            pl.BlockSpec((D_DIM, _VB), lambda i: (0, i)),
            pl.BlockSpec((D_DIM, B_DIM), lambda i: (0, 0)),
        ],
        out_specs=pl.BlockSpec(memory_space=pltpu.MemorySpace.HBM),
        out_shape=jax.ShapeDtypeStruct((VOCAB_SIZE, B_DIM), jnp.float32),
        scratch_shapes=[
            pltpu.VMEM((_K, _VB, B_DIM), jnp.float32),
            pltpu.SemaphoreType.DMA((_K,)),
        ],
        compiler_params=pltpu.CompilerParams(
            vmem_limit_bytes=100 * 1024 * 1024,
        ),
    )(Wt, embt)


def kernel(indices, table, W):
    embt = _sc_gather(indices.astype(jnp.int32), table.T.reshape(-1))
    return _project_t(W.T, embt).T
